# baseline (speedup 1.0000x reference)
import functools
import math

import jax
import jax.numpy as jnp
from jax import lax
from jax.experimental import pallas as pl
from jax.experimental.pallas import tpu as pltpu
from jax.experimental.pallas import tpu_sc as plsc

N = 10000
E = 320000
D = 128
DH = 64
H = 128
G = 64

BN = 2000
NB = N // BN

_INV = 1.0 / math.sqrt(1.0 + 1e-5)


def _filter_body(x_ref, f1w, f1b, f2w, f2b, f3w, f3b, f4w, f4b,
                 m1w, m1b, m2w, m2b, tabs_ref, mask_ref, fx_ref):
    h = x_ref[...]
    h = jnp.maximum(jnp.dot(h, f1w[...], preferred_element_type=jnp.float32) + f1b[...], 0.0)
    h = jnp.maximum(jnp.dot(h, f2w[...], preferred_element_type=jnp.float32) + f2b[...], 0.0)
    h = jnp.maximum(jnp.dot(h, f3w[...], preferred_element_type=jnp.float32) + f3b[...], 0.0)
    fx = jnp.maximum(jnp.dot(h, f4w[...], preferred_element_type=jnp.float32) + f4b[...], 0.0)
    hm = jnp.maximum(jnp.dot(fx, m1w[...], preferred_element_type=jnp.float32) + m1b[...], 0.0)
    logit = jnp.dot(hm, m2w[...], preferred_element_type=jnp.float32) + m2b[...]
    mask = jax.nn.sigmoid(logit)
    mask_ref[...] = mask
    fx_ref[...] = fx
    gcx = fx * mask
    tabs_ref[0, 0] = gcx[:, :DH]
    tabs_ref[0, 1] = gcx[:, DH:]
    tabs_ref[1, 0] = fx[:, :DH]
    tabs_ref[1, 1] = fx[:, DH:]


def _filter_call(x, fw):
    wspec = lambda a: pl.BlockSpec(a.shape, lambda i: (0,) * a.ndim)
    return pl.pallas_call(
        _filter_body,
        grid=(NB,),
        in_specs=[pl.BlockSpec((BN, D), lambda i: (i, 0))] + [wspec(a) for a in fw],
        out_specs=[
            pl.BlockSpec((2, 2, BN, DH), lambda i: (0, 0, i, 0)),
            pl.BlockSpec((BN, 1), lambda i: (i, 0)),
            pl.BlockSpec((BN, D), lambda i: (i, 0)),
        ],
        out_shape=[
            jax.ShapeDtypeStruct((2, 2, N, DH), jnp.float32),
            jax.ShapeDtypeStruct((N, 1), jnp.float32),
            jax.ShapeDtypeStruct((N, D), jnp.float32),
        ],
    )(x, *fw)


def _gin1_body(h1_ref, w1, b1, w2, b2, gm, bt, out_ref):
    e = pl.program_id(0)
    c0 = jnp.where((e == 0) | (e == 2), 1.0, jnp.where(e == 3, -1.0, 0.0))
    c1 = jnp.where((e == 1) | (e == 3), 1.0, 0.0)
    he0 = c0 * h1_ref[0, 0] + c1 * h1_ref[1, 0]
    he1 = c0 * h1_ref[0, 1] + c1 * h1_ref[1, 1]
    z = (jnp.dot(he0, w1[0][:DH], preferred_element_type=jnp.float32)
         + jnp.dot(he1, w1[0][DH:], preferred_element_type=jnp.float32) + b1[0])
    z = jnp.maximum(z, 0.0)
    z = jnp.dot(z, w2[0], preferred_element_type=jnp.float32) + b2[0]
    z = z * (gm[0] * _INV) + bt[0]
    z = jnp.maximum(z, 0.0)
    out_ref[0, 0] = z[:, :DH]
    out_ref[0, 1] = z[:, DH:]


def _gin1_call(h1, w1, b1, w2, b2, gm, bt):
    vec = pl.BlockSpec((1, 1, H), lambda e, i: (e, 0, 0))
    return pl.pallas_call(
        _gin1_body,
        grid=(4, NB),
        in_specs=[
            pl.BlockSpec((2, 2, BN, DH), lambda e, i: (0, 0, i, 0)),
            pl.BlockSpec((1, D, H), lambda e, i: (e, 0, 0)),
            vec,
            pl.BlockSpec((1, H, H), lambda e, i: (e, 0, 0)),
            vec,
            vec,
            vec,
        ],
        out_specs=pl.BlockSpec((1, 2, BN, DH), lambda e, i: (e, 0, i, 0)),
        out_shape=jax.ShapeDtypeStruct((4, 2, N, DH), jnp.float32),
    )(h1, w1, b1, w2, b2, gm, bt)


def _gin2_body(h2_ref, b_ref, w1, b1, w2, b2, gm, bt, s_ref):
    i = pl.program_id(1)
    z = (jnp.dot(h2_ref[0, 0], w1[0][:DH], preferred_element_type=jnp.float32)
         + jnp.dot(h2_ref[0, 1], w1[0][DH:], preferred_element_type=jnp.float32) + b1[0])
    z = jnp.maximum(z, 0.0)
    z = jnp.dot(z, w2[0], preferred_element_type=jnp.float32) + b2[0]
    z = z * (gm[0] * _INV) + bt[0]
    z = jnp.maximum(z, 0.0)
    seg = b_ref[...]
    onehot = (seg == lax.broadcasted_iota(jnp.int32, (BN, G), 1)).astype(jnp.float32)
    sblk = lax.dot_general(onehot, z, (((0,), (0,)), ((), ())),
                           preferred_element_type=jnp.float32)

    @pl.when(i == 0)
    def _():
        s_ref[0] = jnp.zeros_like(s_ref[0])

    s_ref[0] += sblk


def _gin2_call(h2, batch_col, w1, b1, w2, b2, gm, bt):
    vec = pl.BlockSpec((1, 1, H), lambda e, i: (e, 0, 0))
    return pl.pallas_call(
        _gin2_body,
        grid=(4, NB),
        in_specs=[
            pl.BlockSpec((1, 2, BN, DH), lambda e, i: (e, 0, i, 0)),
            pl.BlockSpec((BN, 1), lambda e, i: (i, 0)),
            pl.BlockSpec((1, H, H), lambda e, i: (e, 0, 0)),
            vec,
            pl.BlockSpec((1, H, H), lambda e, i: (e, 0, 0)),
            vec,
            vec,
            vec,
        ],
        out_specs=pl.BlockSpec((1, G, H), lambda e, i: (e, 0, 0)),
        out_shape=jax.ShapeDtypeStruct((4, G, H), jnp.float32),
    )(h2, batch_col, w1, b1, w2, b2, gm, bt)


def _heads_body(s_ref, b_ref, cw, cb, e1w, e1b, e2w, e2b, sw, sb,
                lc_ref, ls_ref, le1_ref, le2_ref, gp_ref, gc_ref, gs_ref):
    seg = b_ref[...]
    onehot = (seg == lax.broadcasted_iota(jnp.int32, (N, G), 1)).astype(jnp.float32)
    ones = jnp.ones((N, 1), jnp.float32)
    cnt = lax.dot_general(onehot, ones, (((0,), (0,)), ((), ())),
                          preferred_element_type=jnp.float32)
    inv = 1.0 / jnp.maximum(cnt, 1.0)
    m_c = s_ref[0] * inv
    m_p = s_ref[1] * inv
    m_ce = s_ref[2] * inv
    m_s = s_ref[3] * inv
    gc_ref[...] = m_c
    gp_ref[...] = m_p
    gs_ref[...] = m_s
    lc_ref[...] = jnp.dot(m_c, cw[...], preferred_element_type=jnp.float32) + cb[...]
    ls_ref[...] = jnp.dot(m_s, sw[...], preferred_element_type=jnp.float32) + sb[...]
    le1_ref[...] = jnp.dot(m_p, e1w[...], preferred_element_type=jnp.float32) + e1b[...]
    le2_ref[...] = jnp.dot(m_ce, e2w[...], preferred_element_type=jnp.float32) + e2b[...]


def _heads_call(s, batch_col, hw):
    full = lambda a: pl.BlockSpec(a.shape, lambda: (0,) * a.ndim)
    return pl.pallas_call(
        _heads_body,
        in_specs=[full(s), full(batch_col)] + [full(a) for a in hw],
        out_specs=[full(jnp.zeros(sh)) for sh in
                   [(G, 2), (G, 1), (G, 2), (G, 2), (G, H), (G, H), (G, H)]],
        out_shape=[
            jax.ShapeDtypeStruct((G, 2), jnp.float32),
            jax.ShapeDtypeStruct((G, 1), jnp.float32),
            jax.ShapeDtypeStruct((G, 2), jnp.float32),
            jax.ShapeDtypeStruct((G, 2), jnp.float32),
            jax.ShapeDtypeStruct((G, H), jnp.float32),
            jax.ShapeDtypeStruct((G, H), jnp.float32),
            jax.ShapeDtypeStruct((G, H), jnp.float32),
        ],
    )(s, batch_col, *hw)


_NC, _NS = 2, 16
_EC = 128
_EPT = 20480
_EPAD = _NS * _EPT - E
_NCH = _EPT // _EC
_RPT = 624
_RC = 104
_TAIL = N - _NS * _RPT
_NBUF = 5


def _make_sc_agg(T):
    mesh = plsc.VectorSubcoreMesh(core_axis_name="c", subcore_axis_name="s")

    @functools.partial(
        pl.kernel,
        out_type=jax.ShapeDtypeStruct((T * 2 * N, DH), jnp.float32),
        mesh=mesh,
        scratch_types=[
            pltpu.VMEM((_NCH, _EC), jnp.int32),
            pltpu.VMEM((_NCH, _EC), jnp.int32),
            pltpu.VMEM((_NBUF, _EC), jnp.int32),
            pltpu.VMEM((_NBUF, _EC, DH), jnp.float32),
            pltpu.VMEM((_RC, DH), jnp.float32),
            pltpu.VMEM((_TAIL, DH), jnp.float32),
            pltpu.VMEM_SHARED((N + 128, DH), jnp.float32),
        ] + [pltpu.SemaphoreType.DMA] * _NBUF,
        compiler_params=pltpu.CompilerParams(use_tc_tiling_on_sc=False),
    )
    def agg(tab_hbm, srcr_hbm, dstr_hbm, out_hbm,
            src_v, dst_v, sidx_v, rows_v, stage_v, tail_v, accum_sh, *sems):
        c = lax.axis_index("c")
        s = lax.axis_index("s")
        pltpu.sync_copy(srcr_hbm.at[s], src_v)
        pltpu.sync_copy(dstr_hbm.at[s], dst_v)
        for t in range(T):
            base_row = (2 * t + c) * N
            for j in range(_RPT // _RC):
                off = s * _RPT + j * _RC
                pltpu.sync_copy(tab_hbm.at[pl.ds(base_row + off, _RC)], stage_v)
                pltpu.sync_copy(stage_v, accum_sh.at[pl.ds(off, _RC)])

            @pl.when(s == 0)
            def _():
                pltpu.sync_copy(tab_hbm.at[pl.ds(base_row + _NS * _RPT, _TAIL)], tail_v)
                pltpu.sync_copy(tail_v, accum_sh.at[pl.ds(_NS * _RPT, _TAIL)])

            plsc.subcore_barrier()

            def issue(b, j):
                for q in range(_EC // 16):
                    sidx_v[b, pl.ds(q * 16, 16)] = src_v[j, pl.ds(q * 16, 16)] + base_row
                pltpu.async_copy(tab_hbm.at[sidx_v.at[b]], rows_v.at[b], sems[b])

            for b in range(_NBUF):
                issue(b, b)

            def group(g, carry):
                for b in range(_NBUF):
                    k = g * _NBUF + b
                    pltpu.make_async_copy(tab_hbm.at[sidx_v.at[b]],
                                          rows_v.at[b], sems[b]).wait()
                    pltpu.sync_copy(rows_v.at[b], accum_sh.at[dst_v.at[k]], add=True)

                    @pl.when(k + _NBUF < _NCH)
                    def _():
                        issue(b, k + _NBUF)
                return carry

            lax.fori_loop(0, _NCH // _NBUF, group, 0)
            plsc.subcore_barrier()
            for j in range(_RPT // _RC):
                off = s * _RPT + j * _RC
                pltpu.sync_copy(accum_sh.at[pl.ds(off, _RC)], stage_v)
                pltpu.sync_copy(stage_v, out_hbm.at[pl.ds(base_row + off, _RC)])

            @pl.when(s == 0)
            def _():
                pltpu.sync_copy(accum_sh.at[pl.ds(_NS * _RPT, _TAIL)], tail_v)
                pltpu.sync_copy(tail_v, out_hbm.at[pl.ds(base_row + _NS * _RPT, _TAIL)])

            if t + 1 < T:
                plsc.subcore_barrier()

    return agg


_sc_agg2 = _make_sc_agg(2)
_sc_agg4 = _make_sc_agg(4)


def _sc_aggregate(tables, src_r, dst_r):
    T = tables.shape[0]
    fn = _sc_agg2 if T == 2 else _sc_agg4
    out = fn(tables.reshape(T * 2 * N, DH), src_r, dst_r)
    return out.reshape(T, 2, N, DH)


def kernel(x, edge_index, batch, params):
    lanes = jnp.arange(_EPAD, dtype=jnp.int32) % 128
    pad_src = lanes
    pad_dst = N + lanes
    src_r = jnp.concatenate([edge_index[0], pad_src]).reshape(_NS, _NCH, _EC)
    dst_r = jnp.concatenate([edge_index[1], pad_dst]).reshape(_NS, _NCH, _EC)
    p = params

    fw = [p["filter"][0]["W"], p["filter"][0]["b"],
          p["filter"][1]["W"], p["filter"][1]["b"],
          p["filter"][2]["W"], p["filter"][2]["b"],
          p["filter"][3]["W"], p["filter"][3]["b"],
          p["masker"][0]["W"], p["masker"][0]["b"],
          p["masker"][1]["W"], p["masker"][1]["b"]]

    encs = [p["clf"], p["env1"], p["env2"], p["senc"]]
    gin_w = {}
    for li in (0, 1):
        gin_w[(li, "l1W")] = jnp.stack([e[li]["l1"]["W"] for e in encs])
        gin_w[(li, "l1b")] = jnp.stack([e[li]["l1"]["b"] for e in encs]).reshape(4, 1, H)
        gin_w[(li, "l2W")] = jnp.stack([e[li]["l2"]["W"] for e in encs])
        gin_w[(li, "l2b")] = jnp.stack([e[li]["l2"]["b"] for e in encs]).reshape(4, 1, H)
        gin_w[(li, "gamma")] = jnp.stack([e[li]["gamma"] for e in encs]).reshape(4, 1, H)
        gin_w[(li, "beta")] = jnp.stack([e[li]["beta"] for e in encs]).reshape(4, 1, H)

    hw = [p["clf_head"]["W"], p["clf_head"]["b"],
          p["env1_head"]["W"], p["env1_head"]["b"],
          p["env2_head"]["W"], p["env2_head"]["b"],
          p["spur_head"]["W"], p["spur_head"]["b"]]

    tabs, mask2d, fx = _filter_call(x, fw)
    mask = mask2d[:, 0]

    h1 = _sc_aggregate(tabs, src_r, dst_r)
    x2 = _gin1_call(h1, gin_w[(0, "l1W")], gin_w[(0, "l1b")],
                    gin_w[(0, "l2W")], gin_w[(0, "l2b")],
                    gin_w[(0, "gamma")], gin_w[(0, "beta")])

    h2 = _sc_aggregate(x2, src_r, dst_r)
    batch_col = batch.reshape(N, 1)
    s = _gin2_call(h2, batch_col, gin_w[(1, "l1W")], gin_w[(1, "l1b")],
                   gin_w[(1, "l2W")], gin_w[(1, "l2b")],
                   gin_w[(1, "gamma")], gin_w[(1, "beta")])

    lc, ls, le1, le2, gp, gc, gs = _heads_call(s, batch_col, hw)
    return (lc, ls, le1, le2, gp, gc, gs, mask, fx)


# speedup vs baseline: 1.0702x; 1.0702x over previous
import functools
import math

import jax
import jax.numpy as jnp
from jax import lax
from jax.experimental import pallas as pl
from jax.experimental.pallas import tpu as pltpu
from jax.experimental.pallas import tpu_sc as plsc

N = 10000
E = 320000
D = 128
DH = 64
H = 128
G = 64

BN = 2000
NB = N // BN

_INV = 1.0 / math.sqrt(1.0 + 1e-5)


def _filter_body(x_ref, f1w, f1b, f2w, f2b, f3w, f3b, f4w, f4b,
                 m1w, m1b, m2w, m2b, tabs_ref, mask_ref, fx_ref):
    h = x_ref[...]
    h = jnp.maximum(jnp.dot(h, f1w[...], preferred_element_type=jnp.float32) + f1b[...], 0.0)
    h = jnp.maximum(jnp.dot(h, f2w[...], preferred_element_type=jnp.float32) + f2b[...], 0.0)
    h = jnp.maximum(jnp.dot(h, f3w[...], preferred_element_type=jnp.float32) + f3b[...], 0.0)
    fx = jnp.maximum(jnp.dot(h, f4w[...], preferred_element_type=jnp.float32) + f4b[...], 0.0)
    hm = jnp.maximum(jnp.dot(fx, m1w[...], preferred_element_type=jnp.float32) + m1b[...], 0.0)
    logit = jnp.dot(hm, m2w[...], preferred_element_type=jnp.float32) + m2b[...]
    mask = jax.nn.sigmoid(logit)
    mask_ref[...] = mask
    fx_ref[...] = fx
    gcx = fx * mask
    tabs_ref[0, 0] = gcx[:, :DH]
    tabs_ref[0, 1] = gcx[:, DH:]
    tabs_ref[1, 0] = fx[:, :DH]
    tabs_ref[1, 1] = fx[:, DH:]


def _filter_call(x, fw):
    wspec = lambda a: pl.BlockSpec(a.shape, lambda i: (0,) * a.ndim)
    return pl.pallas_call(
        _filter_body,
        grid=(NB,),
        in_specs=[pl.BlockSpec((BN, D), lambda i: (i, 0))] + [wspec(a) for a in fw],
        out_specs=[
            pl.BlockSpec((2, 2, BN, DH), lambda i: (0, 0, i, 0)),
            pl.BlockSpec((BN, 1), lambda i: (i, 0)),
            pl.BlockSpec((BN, D), lambda i: (i, 0)),
        ],
        out_shape=[
            jax.ShapeDtypeStruct((2, 2, N, DH), jnp.float32),
            jax.ShapeDtypeStruct((N, 1), jnp.float32),
            jax.ShapeDtypeStruct((N, D), jnp.float32),
        ],
    )(x, *fw)


def _make_gin1(eo):
    def body(h1_ref, w1, b1, w2, b2, gm, bt, out_ref):
        ea = pl.program_id(0) + eo
        c0 = jnp.where((ea == 0) | (ea == 2), 1.0, jnp.where(ea == 3, -1.0, 0.0))
        c1 = jnp.where((ea == 1) | (ea == 3), 1.0, 0.0)
        he0 = c0 * h1_ref[0, 0] + c1 * h1_ref[1, 0]
        he1 = c0 * h1_ref[0, 1] + c1 * h1_ref[1, 1]
        z = (jnp.dot(he0, w1[0][:DH], preferred_element_type=jnp.float32)
             + jnp.dot(he1, w1[0][DH:], preferred_element_type=jnp.float32) + b1[0])
        z = jnp.maximum(z, 0.0)
        z = jnp.dot(z, w2[0], preferred_element_type=jnp.float32) + b2[0]
        z = z * (gm[0] * _INV) + bt[0]
        z = jnp.maximum(z, 0.0)
        out_ref[0, 0] = z[:, :DH]
        out_ref[0, 1] = z[:, DH:]

    vec = pl.BlockSpec((1, 1, H), lambda e, i: (e, 0, 0))

    def call(h1, w1, b1, w2, b2, gm, bt):
        return pl.pallas_call(
            body,
            grid=(2, NB),
            in_specs=[
                pl.BlockSpec((2, 2, BN, DH), lambda e, i: (0, 0, i, 0)),
                pl.BlockSpec((1, D, H), lambda e, i: (e, 0, 0)),
                vec,
                pl.BlockSpec((1, H, H), lambda e, i: (e, 0, 0)),
                vec,
                vec,
                vec,
            ],
            out_specs=pl.BlockSpec((1, 2, BN, DH), lambda e, i: (e, 0, i, 0)),
            out_shape=jax.ShapeDtypeStruct((2, 2, N, DH), jnp.float32),
        )(h1, w1[eo:eo + 2], b1[eo:eo + 2], w2[eo:eo + 2], b2[eo:eo + 2],
          gm[eo:eo + 2], bt[eo:eo + 2])

    return call


_gin1a = _make_gin1(0)
_gin1b = _make_gin1(2)


def _gin2_body(h2_ref, b_ref, w1, b1, w2, b2, gm, bt, s_ref):
    i = pl.program_id(1)
    z = (jnp.dot(h2_ref[0, 0], w1[0][:DH], preferred_element_type=jnp.float32)
         + jnp.dot(h2_ref[0, 1], w1[0][DH:], preferred_element_type=jnp.float32) + b1[0])
    z = jnp.maximum(z, 0.0)
    z = jnp.dot(z, w2[0], preferred_element_type=jnp.float32) + b2[0]
    z = z * (gm[0] * _INV) + bt[0]
    z = jnp.maximum(z, 0.0)
    seg = b_ref[...]
    onehot = (seg == lax.broadcasted_iota(jnp.int32, (BN, G), 1)).astype(jnp.float32)
    sblk = lax.dot_general(onehot, z, (((0,), (0,)), ((), ())),
                           preferred_element_type=jnp.float32)

    @pl.when(i == 0)
    def _():
        s_ref[0] = jnp.zeros_like(s_ref[0])

    s_ref[0] += sblk


def _gin2_call(h2, batch_col, eo, w1, b1, w2, b2, gm, bt):
    vec = pl.BlockSpec((1, 1, H), lambda e, i: (e, 0, 0))
    return pl.pallas_call(
        _gin2_body,
        grid=(2, NB),
        in_specs=[
            pl.BlockSpec((1, 2, BN, DH), lambda e, i: (e, 0, i, 0)),
            pl.BlockSpec((BN, 1), lambda e, i: (i, 0)),
            pl.BlockSpec((1, H, H), lambda e, i: (e, 0, 0)),
            vec,
            pl.BlockSpec((1, H, H), lambda e, i: (e, 0, 0)),
            vec,
            vec,
            vec,
        ],
        out_specs=pl.BlockSpec((1, G, H), lambda e, i: (e, 0, 0)),
        out_shape=jax.ShapeDtypeStruct((2, G, H), jnp.float32),
    )(h2, batch_col, w1[eo:eo + 2], b1[eo:eo + 2], w2[eo:eo + 2],
      b2[eo:eo + 2], gm[eo:eo + 2], bt[eo:eo + 2])


def _heads_body(sa_ref, sb_ref, b_ref, cw, cb, e1w, e1b, e2w, e2b, sw, sb,
                lc_ref, ls_ref, le1_ref, le2_ref, gp_ref, gc_ref, gs_ref):
    seg = b_ref[...]
    onehot = (seg == lax.broadcasted_iota(jnp.int32, (N, G), 1)).astype(jnp.float32)
    ones = jnp.ones((N, 1), jnp.float32)
    cnt = lax.dot_general(onehot, ones, (((0,), (0,)), ((), ())),
                          preferred_element_type=jnp.float32)
    inv = 1.0 / jnp.maximum(cnt, 1.0)
    m_c = sa_ref[0] * inv
    m_p = sa_ref[1] * inv
    m_ce = sb_ref[0] * inv
    m_s = sb_ref[1] * inv
    gc_ref[...] = m_c
    gp_ref[...] = m_p
    gs_ref[...] = m_s
    lc_ref[...] = jnp.dot(m_c, cw[...], preferred_element_type=jnp.float32) + cb[...]
    ls_ref[...] = jnp.dot(m_s, sw[...], preferred_element_type=jnp.float32) + sb[...]
    le1_ref[...] = jnp.dot(m_p, e1w[...], preferred_element_type=jnp.float32) + e1b[...]
    le2_ref[...] = jnp.dot(m_ce, e2w[...], preferred_element_type=jnp.float32) + e2b[...]


def _heads_call(sa, sb, batch_col, hw):
    full = lambda a: pl.BlockSpec(a.shape, lambda: (0,) * a.ndim)
    return pl.pallas_call(
        _heads_body,
        in_specs=[full(sa), full(sb), full(batch_col)] + [full(a) for a in hw],
        out_specs=[full(jnp.zeros(sh)) for sh in
                   [(G, 2), (G, 1), (G, 2), (G, 2), (G, H), (G, H), (G, H)]],
        out_shape=[
            jax.ShapeDtypeStruct((G, 2), jnp.float32),
            jax.ShapeDtypeStruct((G, 1), jnp.float32),
            jax.ShapeDtypeStruct((G, 2), jnp.float32),
            jax.ShapeDtypeStruct((G, 2), jnp.float32),
            jax.ShapeDtypeStruct((G, H), jnp.float32),
            jax.ShapeDtypeStruct((G, H), jnp.float32),
            jax.ShapeDtypeStruct((G, H), jnp.float32),
        ],
    )(sa, sb, batch_col, *hw)


_NC, _NS = 2, 16
_EC = 80
_EPT = E // _NS
_EPAD = _NS * _EPT - E
_NCH = _EPT // _EC
_RPT = 624
_RC = 312
_TAIL = N - _NS * _RPT
_NBUF = 5


def _make_sc_agg(T):
    mesh = plsc.VectorSubcoreMesh(core_axis_name="c", subcore_axis_name="s")

    @functools.partial(
        pl.kernel,
        out_type=jax.ShapeDtypeStruct((T * 2 * N, DH), jnp.float32),
        mesh=mesh,
        scratch_types=[
            pltpu.VMEM((_NCH, _EC), jnp.int32),
            pltpu.VMEM((_NCH, _EC), jnp.int32),
            pltpu.VMEM((_NBUF, _EC), jnp.int32),
            pltpu.VMEM((_NBUF, _EC, DH), jnp.float32),
            pltpu.VMEM((_RC, DH), jnp.float32),
            pltpu.VMEM((_TAIL, DH), jnp.float32),
            pltpu.VMEM_SHARED((N + 128, DH), jnp.float32),
        ] + [pltpu.SemaphoreType.DMA] * _NBUF,
        compiler_params=pltpu.CompilerParams(use_tc_tiling_on_sc=False),
    )
    def agg(tab_hbm, srcr_hbm, dstr_hbm, out_hbm,
            src_v, dst_v, sidx_v, rows_v, stage_v, tail_v, accum_sh, *sems):
        c = lax.axis_index("c")
        s = lax.axis_index("s")
        pltpu.sync_copy(srcr_hbm.at[s], src_v)
        pltpu.sync_copy(dstr_hbm.at[s], dst_v)
        for t in range(T):
            base_row = (2 * t + c) * N
            for j in range(_RPT // _RC):
                off = s * _RPT + j * _RC
                pltpu.sync_copy(tab_hbm.at[pl.ds(base_row + off, _RC)], stage_v)
                pltpu.sync_copy(stage_v, accum_sh.at[pl.ds(off, _RC)])

            @pl.when(s == 0)
            def _():
                pltpu.sync_copy(tab_hbm.at[pl.ds(base_row + _NS * _RPT, _TAIL)], tail_v)
                pltpu.sync_copy(tail_v, accum_sh.at[pl.ds(_NS * _RPT, _TAIL)])

            plsc.subcore_barrier()

            def issue(b, j):
                for q in range(_EC // 16):
                    sidx_v[b, pl.ds(q * 16, 16)] = src_v[j, pl.ds(q * 16, 16)] + base_row
                pltpu.async_copy(tab_hbm.at[sidx_v.at[b]], rows_v.at[b], sems[b])

            for b in range(_NBUF):
                issue(b, b)

            def group(g, carry):
                for b in range(_NBUF):
                    k = g * _NBUF + b
                    pltpu.make_async_copy(tab_hbm.at[sidx_v.at[b]],
                                          rows_v.at[b], sems[b]).wait()
                    pltpu.sync_copy(rows_v.at[b], accum_sh.at[dst_v.at[k]], add=True)

                    @pl.when(k + _NBUF < _NCH)
                    def _():
                        issue(b, k + _NBUF)
                return carry

            lax.fori_loop(0, _NCH // _NBUF, group, 0)
            plsc.subcore_barrier()
            for j in range(_RPT // _RC):
                off = s * _RPT + j * _RC
                pltpu.sync_copy(accum_sh.at[pl.ds(off, _RC)], stage_v)
                pltpu.sync_copy(stage_v, out_hbm.at[pl.ds(base_row + off, _RC)])

            @pl.when(s == 0)
            def _():
                pltpu.sync_copy(accum_sh.at[pl.ds(_NS * _RPT, _TAIL)], tail_v)
                pltpu.sync_copy(tail_v, out_hbm.at[pl.ds(base_row + _NS * _RPT, _TAIL)])

            if t + 1 < T:
                plsc.subcore_barrier()

    return agg


_sc_agg2 = _make_sc_agg(2)


def _sc_aggregate(tables, src_r, dst_r):
    out = _sc_agg2(tables.reshape(2 * 2 * N, DH), src_r, dst_r)
    return out.reshape(2, 2, N, DH)


def kernel(x, edge_index, batch, params):
    lanes = jnp.arange(_EPAD, dtype=jnp.int32) % 128
    pad_src = lanes
    pad_dst = N + lanes
    src_r = jnp.concatenate([edge_index[0], pad_src]).reshape(_NS, _NCH, _EC)
    dst_r = jnp.concatenate([edge_index[1], pad_dst]).reshape(_NS, _NCH, _EC)
    p = params

    fw = [p["filter"][0]["W"], p["filter"][0]["b"],
          p["filter"][1]["W"], p["filter"][1]["b"],
          p["filter"][2]["W"], p["filter"][2]["b"],
          p["filter"][3]["W"], p["filter"][3]["b"],
          p["masker"][0]["W"], p["masker"][0]["b"],
          p["masker"][1]["W"], p["masker"][1]["b"]]

    encs = [p["clf"], p["env1"], p["env2"], p["senc"]]
    gin_w = {}
    for li in (0, 1):
        gin_w[(li, "l1W")] = jnp.stack([e[li]["l1"]["W"] for e in encs])
        gin_w[(li, "l1b")] = jnp.stack([e[li]["l1"]["b"] for e in encs]).reshape(4, 1, H)
        gin_w[(li, "l2W")] = jnp.stack([e[li]["l2"]["W"] for e in encs])
        gin_w[(li, "l2b")] = jnp.stack([e[li]["l2"]["b"] for e in encs]).reshape(4, 1, H)
        gin_w[(li, "gamma")] = jnp.stack([e[li]["gamma"] for e in encs]).reshape(4, 1, H)
        gin_w[(li, "beta")] = jnp.stack([e[li]["beta"] for e in encs]).reshape(4, 1, H)

    hw = [p["clf_head"]["W"], p["clf_head"]["b"],
          p["env1_head"]["W"], p["env1_head"]["b"],
          p["env2_head"]["W"], p["env2_head"]["b"],
          p["spur_head"]["W"], p["spur_head"]["b"]]

    tabs, mask2d, fx = _filter_call(x, fw)
    mask = mask2d[:, 0]

    g1w = (gin_w[(0, "l1W")], gin_w[(0, "l1b")], gin_w[(0, "l2W")],
           gin_w[(0, "l2b")], gin_w[(0, "gamma")], gin_w[(0, "beta")])
    g2w = (gin_w[(1, "l1W")], gin_w[(1, "l1b")], gin_w[(1, "l2W")],
           gin_w[(1, "l2b")], gin_w[(1, "gamma")], gin_w[(1, "beta")])

    h1 = _sc_aggregate(tabs, src_r, dst_r)
    x2a = _gin1a(h1, *g1w)
    h2a = _sc_aggregate(x2a, src_r, dst_r)
    x2b = _gin1b(h1, *g1w)
    h2b = _sc_aggregate(x2b, src_r, dst_r)
    batch_col = batch.reshape(N, 1)
    sa = _gin2_call(h2a, batch_col, 0, *g2w)
    sb = _gin2_call(h2b, batch_col, 2, *g2w)

    lc, ls, le1, le2, gp, gc, gs = _heads_call(sa, sb, batch_col, hw)
    return (lc, ls, le1, le2, gp, gc, gs, mask, fx)


# speedup vs baseline: 1.1642x; 1.0879x over previous
import functools
import math

import jax
import jax.numpy as jnp
from jax import lax
from jax.experimental import pallas as pl
from jax.experimental.pallas import tpu as pltpu
from jax.experimental.pallas import tpu_sc as plsc

N = 10000
E = 320000
D = 128
DH = 64
H = 128
G = 64

BN = 2000
NB = N // BN

_INV = 1.0 / math.sqrt(1.0 + 1e-5)


def _filter_body(x_ref, f1w, f1b, f2w, f2b, f3w, f3b, f4w, f4b,
                 m1w, m1b, m2w, m2b, tabs_ref, mask_ref, fx_ref):
    h = x_ref[...]
    h = jnp.maximum(jnp.dot(h, f1w[...], preferred_element_type=jnp.float32) + f1b[...], 0.0)
    h = jnp.maximum(jnp.dot(h, f2w[...], preferred_element_type=jnp.float32) + f2b[...], 0.0)
    h = jnp.maximum(jnp.dot(h, f3w[...], preferred_element_type=jnp.float32) + f3b[...], 0.0)
    fx = jnp.maximum(jnp.dot(h, f4w[...], preferred_element_type=jnp.float32) + f4b[...], 0.0)
    hm = jnp.maximum(jnp.dot(fx, m1w[...], preferred_element_type=jnp.float32) + m1b[...], 0.0)
    logit = jnp.dot(hm, m2w[...], preferred_element_type=jnp.float32) + m2b[...]
    mask = jax.nn.sigmoid(logit)
    mask_ref[...] = mask
    fx_ref[...] = fx
    gcx = fx * mask
    tabs_ref[0, 0] = gcx[:, :DH]
    tabs_ref[0, 1] = gcx[:, DH:]
    tabs_ref[1, 0] = fx[:, :DH]
    tabs_ref[1, 1] = fx[:, DH:]


def _filter_call(x, fw):
    wspec = lambda a: pl.BlockSpec(a.shape, lambda i: (0,) * a.ndim)
    return pl.pallas_call(
        _filter_body,
        grid=(NB,),
        in_specs=[pl.BlockSpec((BN, D), lambda i: (i, 0))] + [wspec(a) for a in fw],
        out_specs=[
            pl.BlockSpec((2, 2, BN, DH), lambda i: (0, 0, i, 0)),
            pl.BlockSpec((BN, 1), lambda i: (i, 0)),
            pl.BlockSpec((BN, D), lambda i: (i, 0)),
        ],
        out_shape=[
            jax.ShapeDtypeStruct((2, 2, N, DH), jnp.float32),
            jax.ShapeDtypeStruct((N, 1), jnp.float32),
            jax.ShapeDtypeStruct((N, D), jnp.float32),
        ],
    )(x, *fw)


def _make_gin1(eo):
    def body(h1_ref, w1, b1, w2, b2, gm, bt, out_ref):
        ea = pl.program_id(0) + eo
        c0 = jnp.where((ea == 0) | (ea == 2), 1.0, jnp.where(ea == 3, -1.0, 0.0))
        c1 = jnp.where((ea == 1) | (ea == 3), 1.0, 0.0)
        he0 = c0 * h1_ref[0, 0] + c1 * h1_ref[1, 0]
        he1 = c0 * h1_ref[0, 1] + c1 * h1_ref[1, 1]
        z = (jnp.dot(he0, w1[0][:DH], preferred_element_type=jnp.float32)
             + jnp.dot(he1, w1[0][DH:], preferred_element_type=jnp.float32) + b1[0])
        z = jnp.maximum(z, 0.0)
        z = jnp.dot(z, w2[0], preferred_element_type=jnp.float32) + b2[0]
        z = z * (gm[0] * _INV) + bt[0]
        z = jnp.maximum(z, 0.0)
        out_ref[0, 0] = z[:, :DH]
        out_ref[0, 1] = z[:, DH:]

    vec = pl.BlockSpec((1, 1, H), lambda e, i: (e, 0, 0))

    def call(h1, w1, b1, w2, b2, gm, bt):
        return pl.pallas_call(
            body,
            grid=(2, NB),
            in_specs=[
                pl.BlockSpec((2, 2, BN, DH), lambda e, i: (0, 0, i, 0)),
                pl.BlockSpec((1, D, H), lambda e, i: (e, 0, 0)),
                vec,
                pl.BlockSpec((1, H, H), lambda e, i: (e, 0, 0)),
                vec,
                vec,
                vec,
            ],
            out_specs=pl.BlockSpec((1, 2, BN, DH), lambda e, i: (e, 0, i, 0)),
            out_shape=jax.ShapeDtypeStruct((2, 2, N, DH), jnp.float32),
        )(h1, w1[eo:eo + 2], b1[eo:eo + 2], w2[eo:eo + 2], b2[eo:eo + 2],
          gm[eo:eo + 2], bt[eo:eo + 2])

    return call


_gin1a = _make_gin1(0)
_gin1b = _make_gin1(2)


def _gin2_body(h2_ref, b_ref, w1, b1, w2, b2, gm, bt, s_ref):
    i = pl.program_id(1)
    z = (jnp.dot(h2_ref[0, 0], w1[0][:DH], preferred_element_type=jnp.float32)
         + jnp.dot(h2_ref[0, 1], w1[0][DH:], preferred_element_type=jnp.float32) + b1[0])
    z = jnp.maximum(z, 0.0)
    z = jnp.dot(z, w2[0], preferred_element_type=jnp.float32) + b2[0]
    z = z * (gm[0] * _INV) + bt[0]
    z = jnp.maximum(z, 0.0)
    seg = b_ref[...]
    onehot = (seg == lax.broadcasted_iota(jnp.int32, (BN, G), 1)).astype(jnp.float32)
    sblk = lax.dot_general(onehot, z, (((0,), (0,)), ((), ())),
                           preferred_element_type=jnp.float32)

    @pl.when(i == 0)
    def _():
        s_ref[0] = jnp.zeros_like(s_ref[0])

    s_ref[0] += sblk


def _gin2_call(h2, batch_col, eo, w1, b1, w2, b2, gm, bt):
    vec = pl.BlockSpec((1, 1, H), lambda e, i: (e, 0, 0))
    return pl.pallas_call(
        _gin2_body,
        grid=(2, NB),
        in_specs=[
            pl.BlockSpec((1, 2, BN, DH), lambda e, i: (e, 0, i, 0)),
            pl.BlockSpec((BN, 1), lambda e, i: (i, 0)),
            pl.BlockSpec((1, H, H), lambda e, i: (e, 0, 0)),
            vec,
            pl.BlockSpec((1, H, H), lambda e, i: (e, 0, 0)),
            vec,
            vec,
            vec,
        ],
        out_specs=pl.BlockSpec((1, G, H), lambda e, i: (e, 0, 0)),
        out_shape=jax.ShapeDtypeStruct((2, G, H), jnp.float32),
    )(h2, batch_col, w1[eo:eo + 2], b1[eo:eo + 2], w2[eo:eo + 2],
      b2[eo:eo + 2], gm[eo:eo + 2], bt[eo:eo + 2])


def _heads_body(sa_ref, sb_ref, b_ref, cw, cb, e1w, e1b, e2w, e2b, sw, sb,
                lc_ref, ls_ref, le1_ref, le2_ref, gp_ref, gc_ref, gs_ref):
    seg = b_ref[...]
    onehot = (seg == lax.broadcasted_iota(jnp.int32, (N, G), 1)).astype(jnp.float32)
    ones = jnp.ones((N, 1), jnp.float32)
    cnt = lax.dot_general(onehot, ones, (((0,), (0,)), ((), ())),
                          preferred_element_type=jnp.float32)
    inv = 1.0 / jnp.maximum(cnt, 1.0)
    m_c = sa_ref[0] * inv
    m_p = sa_ref[1] * inv
    m_ce = sb_ref[0] * inv
    m_s = sb_ref[1] * inv
    gc_ref[...] = m_c
    gp_ref[...] = m_p
    gs_ref[...] = m_s
    lc_ref[...] = jnp.dot(m_c, cw[...], preferred_element_type=jnp.float32) + cb[...]
    ls_ref[...] = jnp.dot(m_s, sw[...], preferred_element_type=jnp.float32) + sb[...]
    le1_ref[...] = jnp.dot(m_p, e1w[...], preferred_element_type=jnp.float32) + e1b[...]
    le2_ref[...] = jnp.dot(m_ce, e2w[...], preferred_element_type=jnp.float32) + e2b[...]


def _heads_call(sa, sb, batch_col, hw):
    full = lambda a: pl.BlockSpec(a.shape, lambda: (0,) * a.ndim)
    return pl.pallas_call(
        _heads_body,
        in_specs=[full(sa), full(sb), full(batch_col)] + [full(a) for a in hw],
        out_specs=[full(jnp.zeros(sh)) for sh in
                   [(G, 2), (G, 1), (G, 2), (G, 2), (G, H), (G, H), (G, H)]],
        out_shape=[
            jax.ShapeDtypeStruct((G, 2), jnp.float32),
            jax.ShapeDtypeStruct((G, 1), jnp.float32),
            jax.ShapeDtypeStruct((G, 2), jnp.float32),
            jax.ShapeDtypeStruct((G, 2), jnp.float32),
            jax.ShapeDtypeStruct((G, H), jnp.float32),
            jax.ShapeDtypeStruct((G, H), jnp.float32),
            jax.ShapeDtypeStruct((G, H), jnp.float32),
        ],
    )(sa, sb, batch_col, *hw)


_NC, _NS = 2, 16
_EC = 80
_EPT = E // _NS
_EPAD = _NS * _EPT - E
_NCH = _EPT // _EC
_RPT = 624
_RC = 156
_NBK = _RPT // _RC
_NST = 2
_TAIL = N - _NS * _RPT
_NBUF = 5


def _make_sc_agg(T):
    mesh = plsc.VectorSubcoreMesh(core_axis_name="c", subcore_axis_name="s")

    @functools.partial(
        pl.kernel,
        out_type=jax.ShapeDtypeStruct((T * 2 * N, DH), jnp.float32),
        mesh=mesh,
        scratch_types=[
            pltpu.VMEM((_NCH, _EC), jnp.int32),
            pltpu.VMEM((_NCH, _EC), jnp.int32),
            pltpu.VMEM((_NBUF, _EC), jnp.int32),
            pltpu.VMEM((_NBUF, _EC, DH), jnp.float32),
            pltpu.VMEM((_NST, _RC, DH), jnp.float32),
            pltpu.VMEM((_TAIL, DH), jnp.float32),
            pltpu.VMEM_SHARED((N + 128, DH), jnp.float32),
        ] + [pltpu.SemaphoreType.DMA] * (_NBUF + 2 * _NST),
        compiler_params=pltpu.CompilerParams(use_tc_tiling_on_sc=False),
    )
    def agg(tab_hbm, srcr_hbm, dstr_hbm, out_hbm,
            src_v, dst_v, sidx_v, rows_v, stage_v, tail_v, accum_sh, *sems):
        c = lax.axis_index("c")
        s = lax.axis_index("s")
        gsem = sems[:_NBUF]
        psem = sems[_NBUF:_NBUF + _NST]
        wsem = sems[_NBUF + _NST:]
        pltpu.sync_copy(srcr_hbm.at[s], src_v)
        pltpu.sync_copy(dstr_hbm.at[s], dst_v)
        for t in range(T):
            base_row = (2 * t + c) * N
            for j in range(_NST):
                off = s * _RPT + j * _RC
                if t > 0:
                    pltpu.make_async_copy(
                        stage_v.at[j], out_hbm.at[pl.ds(base_row + off, _RC)],
                        wsem[j]).wait()
                pltpu.async_copy(tab_hbm.at[pl.ds(base_row + off, _RC)],
                                 stage_v.at[j], psem[j])
            for j in range(_NBK):
                b = j % _NST
                off = s * _RPT + j * _RC
                pltpu.make_async_copy(tab_hbm.at[pl.ds(base_row + off, _RC)],
                                      stage_v.at[b], psem[b]).wait()
                pltpu.sync_copy(stage_v.at[b], accum_sh.at[pl.ds(off, _RC)])
                if j + _NST < _NBK:
                    off2 = s * _RPT + (j + _NST) * _RC
                    pltpu.async_copy(tab_hbm.at[pl.ds(base_row + off2, _RC)],
                                     stage_v.at[b], psem[b])

            @pl.when(s == 0)
            def _():
                pltpu.sync_copy(tab_hbm.at[pl.ds(base_row + _NS * _RPT, _TAIL)], tail_v)
                pltpu.sync_copy(tail_v, accum_sh.at[pl.ds(_NS * _RPT, _TAIL)])

            plsc.subcore_barrier()

            def issue(b, j):
                for q in range(_EC // 16):
                    sidx_v[b, pl.ds(q * 16, 16)] = src_v[j, pl.ds(q * 16, 16)] + base_row
                pltpu.async_copy(tab_hbm.at[sidx_v.at[b]], rows_v.at[b], sems[b])

            for b in range(_NBUF):
                issue(b, b)

            def group(g, carry):
                for b in range(_NBUF):
                    k = g * _NBUF + b
                    pltpu.make_async_copy(tab_hbm.at[sidx_v.at[b]],
                                          rows_v.at[b], sems[b]).wait()
                    pltpu.sync_copy(rows_v.at[b], accum_sh.at[dst_v.at[k]], add=True)

                    @pl.when(k + _NBUF < _NCH)
                    def _():
                        issue(b, k + _NBUF)
                return carry

            lax.fori_loop(0, _NCH // _NBUF, group, 0)
            plsc.subcore_barrier()
            for j in range(_NBK):
                b = j % _NST
                off = s * _RPT + j * _RC
                if j >= _NST:
                    off0 = s * _RPT + (j - _NST) * _RC
                    pltpu.make_async_copy(
                        stage_v.at[b], out_hbm.at[pl.ds(base_row + off0, _RC)],
                        wsem[b]).wait()
                pltpu.sync_copy(accum_sh.at[pl.ds(off, _RC)], stage_v.at[b])
                pltpu.async_copy(stage_v.at[b],
                                 out_hbm.at[pl.ds(base_row + off, _RC)], wsem[b])

            @pl.when(s == 0)
            def _():
                pltpu.sync_copy(accum_sh.at[pl.ds(_NS * _RPT, _TAIL)], tail_v)
                pltpu.sync_copy(tail_v, out_hbm.at[pl.ds(base_row + _NS * _RPT, _TAIL)])

        last_base = (2 * (T - 1) + c) * N
        for j in range(_NBK - _NST, _NBK):
            b = j % _NST
            off = s * _RPT + j * _RC
            pltpu.make_async_copy(stage_v.at[b],
                                  out_hbm.at[pl.ds(last_base + off, _RC)],
                                  wsem[b]).wait()

    return agg


_sc_agg2 = _make_sc_agg(2)


def _sc_aggregate(tables, src_r, dst_r):
    out = _sc_agg2(tables.reshape(2 * 2 * N, DH), src_r, dst_r)
    return out.reshape(2, 2, N, DH)


def kernel(x, edge_index, batch, params):
    lanes = jnp.arange(_EPAD, dtype=jnp.int32) % 128
    pad_src = lanes
    pad_dst = N + lanes
    src_r = jnp.concatenate([edge_index[0], pad_src]).reshape(_NS, _NCH, _EC)
    dst_r = jnp.concatenate([edge_index[1], pad_dst]).reshape(_NS, _NCH, _EC)
    p = params

    fw = [p["filter"][0]["W"], p["filter"][0]["b"],
          p["filter"][1]["W"], p["filter"][1]["b"],
          p["filter"][2]["W"], p["filter"][2]["b"],
          p["filter"][3]["W"], p["filter"][3]["b"],
          p["masker"][0]["W"], p["masker"][0]["b"],
          p["masker"][1]["W"], p["masker"][1]["b"]]

    encs = [p["clf"], p["env1"], p["env2"], p["senc"]]
    gin_w = {}
    for li in (0, 1):
        gin_w[(li, "l1W")] = jnp.stack([e[li]["l1"]["W"] for e in encs])
        gin_w[(li, "l1b")] = jnp.stack([e[li]["l1"]["b"] for e in encs]).reshape(4, 1, H)
        gin_w[(li, "l2W")] = jnp.stack([e[li]["l2"]["W"] for e in encs])
        gin_w[(li, "l2b")] = jnp.stack([e[li]["l2"]["b"] for e in encs]).reshape(4, 1, H)
        gin_w[(li, "gamma")] = jnp.stack([e[li]["gamma"] for e in encs]).reshape(4, 1, H)
        gin_w[(li, "beta")] = jnp.stack([e[li]["beta"] for e in encs]).reshape(4, 1, H)

    hw = [p["clf_head"]["W"], p["clf_head"]["b"],
          p["env1_head"]["W"], p["env1_head"]["b"],
          p["env2_head"]["W"], p["env2_head"]["b"],
          p["spur_head"]["W"], p["spur_head"]["b"]]

    tabs, mask2d, fx = _filter_call(x, fw)
    mask = mask2d[:, 0]

    g1w = (gin_w[(0, "l1W")], gin_w[(0, "l1b")], gin_w[(0, "l2W")],
           gin_w[(0, "l2b")], gin_w[(0, "gamma")], gin_w[(0, "beta")])
    g2w = (gin_w[(1, "l1W")], gin_w[(1, "l1b")], gin_w[(1, "l2W")],
           gin_w[(1, "l2b")], gin_w[(1, "gamma")], gin_w[(1, "beta")])

    h1 = _sc_aggregate(tabs, src_r, dst_r)
    x2a = _gin1a(h1, *g1w)
    h2a = _sc_aggregate(x2a, src_r, dst_r)
    x2b = _gin1b(h1, *g1w)
    h2b = _sc_aggregate(x2b, src_r, dst_r)
    batch_col = batch.reshape(N, 1)
    sa = _gin2_call(h2a, batch_col, 0, *g2w)
    sb = _gin2_call(h2b, batch_col, 2, *g2w)

    lc, ls, le1, le2, gp, gc, gs = _heads_call(sa, sb, batch_col, hw)
    return (lc, ls, le1, le2, gp, gc, gs, mask, fx)
